# baseline (speedup 1.0000x reference)
import jax
import jax.numpy as jnp
from jax import lax
from jax.experimental import pallas as pl
from jax.experimental.pallas import tpu as pltpu
from jax.experimental.pallas import tpu_sc as plsc

N = 10000
D = 256
E = 160000

NC = 2
NS = 16
ACC_ROWS = 10240
STRIPE = ACC_ROWS // NS
CHUNK = 40
EPW = E // NS
NCHUNK = EPW // CHUNK
NBUF = 4
NMAIN = NCHUNK // NBUF
NTAIL = NCHUNK % NBUF
DEG_ROWS = 80

ROWBLK = 1000

import dataclasses as _dataclasses
import functools as _functools


@_functools.cache
def _sc_compiler_params():
    cp = pltpu.CompilerParams()
    if "needs_layout_passes" in pltpu.CompilerParams.__dataclass_fields__:
        cp = _dataclasses.replace(cp, needs_layout_passes=False)
    return cp


@_functools.cache
def _vector_mesh():
    return plsc.VectorSubcoreMesh(core_axis_name="core",
                                  subcore_axis_name="subcore",
                                  num_cores=NC, num_subcores=NS)


def _deg_body(edge_hbm, iota_hbm, deg_hbm, dstv, part, iotav, spdeg):
    c = lax.axis_index("core")
    s = lax.axis_index("subcore")
    stripe = 8

    @pl.loop(0, DEG_ROWS)
    def _(i):
        for j in range(8):
            part[i, pl.ds(j * 16, 16)] = jnp.zeros((16,), jnp.float32)

    @pl.when(s < DEG_ROWS // stripe)
    def _():
        pltpu.sync_copy(part.at[pl.ds(0, stripe)],
                        spdeg.at[pl.ds(s * stripe, stripe)])

    pltpu.sync_copy(iota_hbm, iotav)
    pltpu.sync_copy(edge_hbm.at[pl.ds(E + s * EPW, EPW)], dstv)

    @pl.loop(0, EPW // 16)
    def _(i):
        idx = dstv[pl.ds(i * 16, 16)]
        row = lax.shift_right_logical(idx, 7)
        col = jnp.bitwise_and(idx, 127)
        plsc.addupdate_scatter(part, [row, col], jnp.ones((16,), jnp.float32))

    plsc.subcore_barrier()
    pltpu.sync_copy(part, spdeg.at[iotav], add=True)
    plsc.subcore_barrier()

    @pl.when((c == 0) & (s < DEG_ROWS // stripe))
    def _():
        pltpu.sync_copy(spdeg.at[pl.ds(s * stripe, stripe)],
                        deg_hbm.at[pl.ds(s * stripe, stripe)])


def _compute_deg(edges, iota80):
    kfn = pl.kernel(
        _deg_body,
        out_type=jax.ShapeDtypeStruct((DEG_ROWS, 128), jnp.float32),
        mesh=_vector_mesh(),
        compiler_params=_sc_compiler_params(),
        scratch_types=[
            pltpu.VMEM((EPW,), jnp.int32),
            pltpu.VMEM((DEG_ROWS, 128), jnp.float32),
            pltpu.VMEM((DEG_ROWS,), jnp.int32),
            pltpu.VMEM_SHARED((DEG_ROWS, 128), jnp.float32),
        ],
    )
    return kfn(edges, iota80)


def _agg_body(edge_hbm, hs2_hbm, out_hbm,
              srcv, dstloc, rows0, rows1, rows2, rows3, zblk, acc,
              sem0, sem1, sem2, sem3):
    c = lax.axis_index("core")
    s = lax.axis_index("subcore")
    bufs = (rows0, rows1, rows2, rows3)
    sems = (sem0, sem1, sem2, sem3)

    pltpu.sync_copy(edge_hbm.at[pl.ds(s * EPW, EPW)], srcv)

    @pl.loop(0, EPW // 16)
    def _(i):
        v = srcv[pl.ds(i * 16, 16)]
        srcv[pl.ds(i * 16, 16)] = v + v + c

    @pl.loop(0, 8)
    def _(i):
        for j in range(128 // 16):
            zblk[i, pl.ds(j * 16, 16)] = jnp.zeros((16,), jnp.float32)

    def start_gather(i, b):
        pltpu.async_copy(hs2_hbm.at[srcv.at[pl.ds(i * CHUNK, CHUNK)]],
                         bufs[b], sems[b])

    def wait_gather(i, b):
        pltpu.make_async_copy(hs2_hbm.at[srcv.at[pl.ds(i * CHUNK, CHUNK)]],
                              bufs[b], sems[b]).wait()

    def start_scatter(i, b):
        pltpu.async_copy(bufs[b], acc.at[dstloc.at[pl.ds(i * CHUNK, CHUNK)]],
                         sems[b], add=True)

    def wait_scatter(i, b):
        pltpu.make_async_copy(bufs[b],
                              acc.at[dstloc.at[pl.ds(i * CHUNK, CHUNK)]],
                              sems[b]).wait()

    @pl.loop(0, STRIPE // 8)
    def _(k):
        pltpu.sync_copy(zblk, acc.at[pl.ds(s * STRIPE + k * 8, 8)])

    pltpu.sync_copy(edge_hbm.at[pl.ds(E + s * EPW, EPW)], dstloc)
    plsc.subcore_barrier()

    for b in range(NBUF):
        start_gather(b, b)

    @pl.loop(0, NMAIN)
    def _(k):
        i = k * NBUF
        for b in range(NBUF):
            wait_gather(i + b, b)
            start_scatter(i + b, b)
        for b in range(NBUF):
            wait_scatter(i + b, b)

            @pl.when(i + NBUF + b < NCHUNK)
            def _(b=b, i=i):
                start_gather(i + NBUF + b, b)

    for b in range(NTAIL):
        wait_gather(NMAIN * NBUF + b, b)
        start_scatter(NMAIN * NBUF + b, b)
    for b in range(NTAIL):
        wait_scatter(NMAIN * NBUF + b, b)

    plsc.subcore_barrier()

    pltpu.sync_copy(acc.at[pl.ds(s * STRIPE, STRIPE)],
                    out_hbm.at[c, pl.ds(s * STRIPE, STRIPE)])


def _aggregate(edges, hs2):
    kfn = pl.kernel(
        _agg_body,
        out_type=jax.ShapeDtypeStruct((NC, ACC_ROWS, 128), jnp.float32),
        mesh=_vector_mesh(),
        compiler_params=_sc_compiler_params(),
        scratch_types=[
            pltpu.VMEM((EPW,), jnp.int32),
            pltpu.VMEM((EPW,), jnp.int32),
            pltpu.VMEM((CHUNK, 128), jnp.float32),
            pltpu.VMEM((CHUNK, 128), jnp.float32),
            pltpu.VMEM((CHUNK, 128), jnp.float32),
            pltpu.VMEM((CHUNK, 128), jnp.float32),
            pltpu.VMEM((8, 128), jnp.float32),
            pltpu.VMEM_SHARED((ACC_ROWS, 128), jnp.float32),
            pltpu.SemaphoreType.DMA,
            pltpu.SemaphoreType.DMA,
            pltpu.SemaphoreType.DMA,
            pltpu.SemaphoreType.DMA,
        ],
    )
    return kfn(edges, hs2)


def _k1_body(deg_ref, x_ref, w_ref, hs_ref, dis_ref):
    dis = lax.rsqrt(deg_ref[...] + 1.0)
    h = jnp.dot(x_ref[...], w_ref[...], preferred_element_type=jnp.float32)
    hs_ref[...] = (h * dis).reshape(2 * ROWBLK, 128)
    dis_ref[...] = dis


def _k2_body(a0_ref, a1_ref, hs_ref, dis_ref, b_ref, w_ref, hs2_ref):
    dis = dis_ref[...]
    agg = jnp.concatenate([a0_ref[0], a1_ref[0]], axis=1)
    hs = hs_ref[...].reshape(ROWBLK, D)
    t = jnp.maximum(dis * (agg + hs) + b_ref[...], 0.0)
    h2 = jnp.dot(t, w_ref[...], preferred_element_type=jnp.float32)
    hs2_ref[...] = (dis * h2).reshape(2 * ROWBLK, 128)


def _k3_body(a0_ref, a1_ref, hs_ref, dis_ref, b_ref, o_ref):
    agg = jnp.concatenate([a0_ref[0], a1_ref[0]], axis=1)
    hs = hs_ref[...].reshape(ROWBLK, D)
    o_ref[...] = dis_ref[...] * (agg + hs) + b_ref[...]


_NBLK = N // ROWBLK

def _row_spec(width):
    return pl.BlockSpec((ROWBLK, width), lambda i: (i, 0))

def _hs_spec():
    return pl.BlockSpec((2 * ROWBLK, 128), lambda i: (i, 0))

def _agg_spec(core):
    return pl.BlockSpec((1, ROWBLK, 128), lambda i: (core, i, 0))

def _full_spec(r, cols):
    return pl.BlockSpec((r, cols), lambda i: (0, 0))


def _tc_k1(deg, x, W1):
    return pl.pallas_call(
        _k1_body,
        grid=(_NBLK,),
        in_specs=[_row_spec(1), _row_spec(D), _full_spec(D, D)],
        out_specs=[_hs_spec(), _row_spec(1)],
        out_shape=[jax.ShapeDtypeStruct((2 * N, 128), jnp.float32),
                   jax.ShapeDtypeStruct((N, 1), jnp.float32)],
    )(deg, x, W1)


def _tc_k2(agg, hs2, dis, b1, W2):
    return pl.pallas_call(
        _k2_body,
        grid=(_NBLK,),
        in_specs=[_agg_spec(0), _agg_spec(1), _hs_spec(), _row_spec(1),
                  _full_spec(1, D), _full_spec(D, D)],
        out_specs=_hs_spec(),
        out_shape=jax.ShapeDtypeStruct((2 * N, 128), jnp.float32),
    )(agg, agg, hs2, dis, b1, W2)


def _tc_k3(agg, hs2, dis, b2):
    return pl.pallas_call(
        _k3_body,
        grid=(_NBLK,),
        in_specs=[_agg_spec(0), _agg_spec(1), _hs_spec(), _row_spec(1),
                  _full_spec(1, D)],
        out_specs=_row_spec(D),
        out_shape=jax.ShapeDtypeStruct((N, D), jnp.float32),
    )(agg, agg, hs2, dis, b2)


def kernel(x, edge_index, W1, b1, W2, b2):
    edges = edge_index.astype(jnp.int32).reshape(2 * E)
    iota80 = jnp.arange(DEG_ROWS, dtype=jnp.int32)
    b1r = b1.reshape(1, D)
    b2r = b2.reshape(1, D)

    deg = _compute_deg(edges, iota80).reshape(-1)[:N].reshape(N, 1)
    hs1, dis = _tc_k1(deg, x, W1)
    agg1 = _aggregate(edges, hs1)
    hs2 = _tc_k2(agg1, hs1, dis, b1r, W2)
    agg2 = _aggregate(edges, hs2)
    return _tc_k3(agg2, hs2, dis, b2r)


# speedup vs baseline: 1.0092x; 1.0092x over previous
import jax
import jax.numpy as jnp
from jax import lax
from jax.experimental import pallas as pl
from jax.experimental.pallas import tpu as pltpu
from jax.experimental.pallas import tpu_sc as plsc

N = 10000
D = 256
E = 160000

NC = 2
NS = 16
ACC_ROWS = 10240
STRIPE = ACC_ROWS // NS
CHUNK = 40
EPW = E // NS
NCHUNK = EPW // CHUNK
NBUF = 4
NMAIN = NCHUNK // NBUF
NTAIL = NCHUNK % NBUF
DEG_ROWS = 80

ROWBLK = 2000

import dataclasses as _dataclasses
import functools as _functools


@_functools.cache
def _sc_compiler_params():
    cp = pltpu.CompilerParams()
    if "needs_layout_passes" in pltpu.CompilerParams.__dataclass_fields__:
        cp = _dataclasses.replace(cp, needs_layout_passes=False)
    return cp


@_functools.cache
def _vector_mesh():
    return plsc.VectorSubcoreMesh(core_axis_name="core",
                                  subcore_axis_name="subcore",
                                  num_cores=NC, num_subcores=NS)


def _deg_body(edge_hbm, iota_hbm, deg_hbm, dstv, part, iotav, spdeg):
    c = lax.axis_index("core")
    s = lax.axis_index("subcore")
    stripe = 8

    @pl.loop(0, DEG_ROWS)
    def _(i):
        for j in range(8):
            part[i, pl.ds(j * 16, 16)] = jnp.zeros((16,), jnp.float32)

    @pl.when(s < DEG_ROWS // stripe)
    def _():
        pltpu.sync_copy(part.at[pl.ds(0, stripe)],
                        spdeg.at[pl.ds(s * stripe, stripe)])

    pltpu.sync_copy(iota_hbm, iotav)
    pltpu.sync_copy(edge_hbm.at[pl.ds(E + s * EPW, EPW)], dstv)

    @pl.loop(0, EPW // 16)
    def _(i):
        idx = dstv[pl.ds(i * 16, 16)]
        row = lax.shift_right_logical(idx, 7)
        col = jnp.bitwise_and(idx, 127)
        plsc.addupdate_scatter(part, [row, col], jnp.ones((16,), jnp.float32))

    plsc.subcore_barrier()
    pltpu.sync_copy(part, spdeg.at[iotav], add=True)
    plsc.subcore_barrier()

    @pl.when((c == 0) & (s < DEG_ROWS // stripe))
    def _():
        pltpu.sync_copy(spdeg.at[pl.ds(s * stripe, stripe)],
                        deg_hbm.at[pl.ds(s * stripe, stripe)])


def _compute_deg(edges, iota80):
    kfn = pl.kernel(
        _deg_body,
        out_type=jax.ShapeDtypeStruct((DEG_ROWS, 128), jnp.float32),
        mesh=_vector_mesh(),
        compiler_params=_sc_compiler_params(),
        scratch_types=[
            pltpu.VMEM((EPW,), jnp.int32),
            pltpu.VMEM((DEG_ROWS, 128), jnp.float32),
            pltpu.VMEM((DEG_ROWS,), jnp.int32),
            pltpu.VMEM_SHARED((DEG_ROWS, 128), jnp.float32),
        ],
    )
    return kfn(edges, iota80)


def _agg_body(edge_hbm, hs2_hbm, out_hbm,
              srcv, dstloc, rows0, rows1, rows2, rows3, zblk, acc,
              sem0, sem1, sem2, sem3):
    c = lax.axis_index("core")
    s = lax.axis_index("subcore")
    bufs = (rows0, rows1, rows2, rows3)
    sems = (sem0, sem1, sem2, sem3)

    pltpu.sync_copy(edge_hbm.at[pl.ds(s * EPW, EPW)], srcv)

    @pl.loop(0, EPW // 16)
    def _(i):
        v = srcv[pl.ds(i * 16, 16)]
        srcv[pl.ds(i * 16, 16)] = v + v + c

    @pl.loop(0, 8)
    def _(i):
        for j in range(128 // 16):
            zblk[i, pl.ds(j * 16, 16)] = jnp.zeros((16,), jnp.float32)

    def start_gather(i, b):
        pltpu.async_copy(hs2_hbm.at[srcv.at[pl.ds(i * CHUNK, CHUNK)]],
                         bufs[b], sems[b])

    def wait_gather(i, b):
        pltpu.make_async_copy(hs2_hbm.at[srcv.at[pl.ds(i * CHUNK, CHUNK)]],
                              bufs[b], sems[b]).wait()

    def start_scatter(i, b):
        pltpu.async_copy(bufs[b], acc.at[dstloc.at[pl.ds(i * CHUNK, CHUNK)]],
                         sems[b], add=True)

    def wait_scatter(i, b):
        pltpu.make_async_copy(bufs[b],
                              acc.at[dstloc.at[pl.ds(i * CHUNK, CHUNK)]],
                              sems[b]).wait()

    @pl.loop(0, STRIPE // 8)
    def _(k):
        pltpu.sync_copy(zblk, acc.at[pl.ds(s * STRIPE + k * 8, 8)])

    pltpu.sync_copy(edge_hbm.at[pl.ds(E + s * EPW, EPW)], dstloc)
    plsc.subcore_barrier()

    for b in range(NBUF):
        start_gather(b, b)

    @pl.loop(0, NMAIN)
    def _(k):
        i = k * NBUF
        for b in range(NBUF):
            wait_gather(i + b, b)
            start_scatter(i + b, b)
        for b in range(NBUF):
            wait_scatter(i + b, b)

            @pl.when(i + NBUF + b < NCHUNK)
            def _(b=b, i=i):
                start_gather(i + NBUF + b, b)

    for b in range(NTAIL):
        wait_gather(NMAIN * NBUF + b, b)
        start_scatter(NMAIN * NBUF + b, b)
    for b in range(NTAIL):
        wait_scatter(NMAIN * NBUF + b, b)

    plsc.subcore_barrier()

    pltpu.sync_copy(acc.at[pl.ds(s * STRIPE, STRIPE)],
                    out_hbm.at[c, pl.ds(s * STRIPE, STRIPE)])


def _aggregate(edges, hs2):
    kfn = pl.kernel(
        _agg_body,
        out_type=jax.ShapeDtypeStruct((NC, ACC_ROWS, 128), jnp.float32),
        mesh=_vector_mesh(),
        compiler_params=_sc_compiler_params(),
        scratch_types=[
            pltpu.VMEM((EPW,), jnp.int32),
            pltpu.VMEM((EPW,), jnp.int32),
            pltpu.VMEM((CHUNK, 128), jnp.float32),
            pltpu.VMEM((CHUNK, 128), jnp.float32),
            pltpu.VMEM((CHUNK, 128), jnp.float32),
            pltpu.VMEM((CHUNK, 128), jnp.float32),
            pltpu.VMEM((8, 128), jnp.float32),
            pltpu.VMEM_SHARED((ACC_ROWS, 128), jnp.float32),
            pltpu.SemaphoreType.DMA,
            pltpu.SemaphoreType.DMA,
            pltpu.SemaphoreType.DMA,
            pltpu.SemaphoreType.DMA,
        ],
    )
    return kfn(edges, hs2)


def _k1a_body(x_ref, w_ref, h_ref):
    h = jnp.dot(x_ref[...], w_ref[...], preferred_element_type=jnp.float32)
    h_ref[...] = h.reshape(2 * ROWBLK, 128)


def _k1b_body(deg_ref, h_ref, hs_ref, dis_ref):
    dis = lax.rsqrt(deg_ref[...] + 1.0)
    h = h_ref[...].reshape(ROWBLK, D)
    hs_ref[...] = (h * dis).reshape(2 * ROWBLK, 128)
    dis_ref[...] = dis


def _k2_body(a0_ref, a1_ref, hs_ref, dis_ref, b_ref, w_ref, hs2_ref):
    dis = dis_ref[...]
    agg = jnp.concatenate([a0_ref[0], a1_ref[0]], axis=1)
    hs = hs_ref[...].reshape(ROWBLK, D)
    t = jnp.maximum(dis * (agg + hs) + b_ref[...], 0.0)
    h2 = jnp.dot(t, w_ref[...], preferred_element_type=jnp.float32)
    hs2_ref[...] = (dis * h2).reshape(2 * ROWBLK, 128)


def _k3_body(a0_ref, a1_ref, hs_ref, dis_ref, b_ref, o_ref):
    agg = jnp.concatenate([a0_ref[0], a1_ref[0]], axis=1)
    hs = hs_ref[...].reshape(ROWBLK, D)
    o_ref[...] = dis_ref[...] * (agg + hs) + b_ref[...]


_NBLK = N // ROWBLK

def _row_spec(width):
    return pl.BlockSpec((ROWBLK, width), lambda i: (i, 0))

def _hs_spec():
    return pl.BlockSpec((2 * ROWBLK, 128), lambda i: (i, 0))

def _agg_spec(core):
    return pl.BlockSpec((1, ROWBLK, 128), lambda i: (core, i, 0))

def _full_spec(r, cols):
    return pl.BlockSpec((r, cols), lambda i: (0, 0))


def _tc_k1a(x, W1):
    return pl.pallas_call(
        _k1a_body,
        grid=(_NBLK,),
        in_specs=[_row_spec(D), _full_spec(D, D)],
        out_specs=_hs_spec(),
        out_shape=jax.ShapeDtypeStruct((2 * N, 128), jnp.float32),
    )(x, W1)


def _tc_k1b(deg, h1):
    return pl.pallas_call(
        _k1b_body,
        grid=(_NBLK,),
        in_specs=[_row_spec(1), _hs_spec()],
        out_specs=[_hs_spec(), _row_spec(1)],
        out_shape=[jax.ShapeDtypeStruct((2 * N, 128), jnp.float32),
                   jax.ShapeDtypeStruct((N, 1), jnp.float32)],
    )(deg, h1)


def _tc_k2(agg, hs2, dis, b1, W2):
    return pl.pallas_call(
        _k2_body,
        grid=(_NBLK,),
        in_specs=[_agg_spec(0), _agg_spec(1), _hs_spec(), _row_spec(1),
                  _full_spec(1, D), _full_spec(D, D)],
        out_specs=_hs_spec(),
        out_shape=jax.ShapeDtypeStruct((2 * N, 128), jnp.float32),
    )(agg, agg, hs2, dis, b1, W2)


def _tc_k3(agg, hs2, dis, b2):
    return pl.pallas_call(
        _k3_body,
        grid=(_NBLK,),
        in_specs=[_agg_spec(0), _agg_spec(1), _hs_spec(), _row_spec(1),
                  _full_spec(1, D)],
        out_specs=_row_spec(D),
        out_shape=jax.ShapeDtypeStruct((N, D), jnp.float32),
    )(agg, agg, hs2, dis, b2)


def kernel(x, edge_index, W1, b1, W2, b2):
    edges = edge_index.astype(jnp.int32).reshape(2 * E)
    iota80 = jnp.arange(DEG_ROWS, dtype=jnp.int32)
    b1r = b1.reshape(1, D)
    b2r = b2.reshape(1, D)

    deg = _compute_deg(edges, iota80).reshape(-1)[:N].reshape(N, 1)
    h1 = _tc_k1a(x, W1)
    hs1, dis = _tc_k1b(deg, h1)
    agg1 = _aggregate(edges, hs1)
    hs2 = _tc_k2(agg1, hs1, dis, b1r, W2)
    agg2 = _aggregate(edges, hs2)
    return _tc_k3(agg2, hs2, dis, b2r)
